# baseline (speedup 1.0000x reference)
import functools

import jax
import jax.numpy as jnp
from jax import lax
from jax.experimental import pallas as pl
from jax.experimental.pallas import tpu as pltpu
from jax.experimental.pallas import tpu_sc as plsc

_DIM = 1024
_NTOK = 8192
_NW = 32
_TPW = _NTOK // _NW
_CH = 32
_NCHUNK = _TPW // _CH
_L = 2048
_LANES = 16


def _emb_body(ids_hbm, tok_hbm, pos_hbm, out_hbm, idx_v, rows_v, pos_v,
              gsem, psem):
    nc = 2
    wid = lax.axis_index("s") * nc + lax.axis_index("c")
    base = wid * _TPW
    l0 = lax.rem(base, _L)
    pltpu.sync_copy(ids_hbm.at[pl.ds(base, _TPW)], idx_v)

    def chunk_body(i, carry):
        off = i * _CH
        gat = pltpu.async_copy(tok_hbm.at[idx_v.at[pl.ds(off, _CH)]],
                               rows_v, gsem)
        pcp = pltpu.async_copy(pos_hbm.at[pl.ds(l0 + off, _CH)], pos_v, psem)
        pcp.wait()
        gat.wait()

        def row_body(r, c2):
            for c in range(_DIM // _LANES):
                sl = pl.ds(c * _LANES, _LANES)
                plsc.addupdate(rows_v.at[r, sl], pos_v[r, sl])
            return c2

        lax.fori_loop(0, _CH, row_body, 0)
        pltpu.sync_copy(rows_v, out_hbm.at[pl.ds(base + off, _CH)])
        return carry

    lax.fori_loop(0, _NCHUNK, chunk_body, 0)


@jax.jit
def _emb(ids_flat, tok_table, pos_table):
    mesh = plsc.VectorSubcoreMesh(core_axis_name="c", subcore_axis_name="s")
    return pl.kernel(
        _emb_body,
        out_type=jax.ShapeDtypeStruct((_NTOK, _DIM), jnp.float32),
        mesh=mesh,
        scratch_types=[
            pltpu.VMEM((_TPW,), jnp.int32),
            pltpu.VMEM((_CH, _DIM), jnp.float32),
            pltpu.VMEM((_CH, _DIM), jnp.float32),
            pltpu.SemaphoreType.DMA,
            pltpu.SemaphoreType.DMA,
        ],
    )(ids_flat, tok_table, pos_table)


def kernel(token_ids, tok_table, pos_table):
    b, l = token_ids.shape
    dim = tok_table.shape[1]
    ids_flat = token_ids.reshape(b * l).astype(jnp.int32)
    out = _emb(ids_flat, tok_table, pos_table)
    return out.reshape(b, l, dim)


# speedup vs baseline: 1.6139x; 1.6139x over previous
import jax
import jax.numpy as jnp
from jax import lax
from jax.experimental import pallas as pl
from jax.experimental.pallas import tpu as pltpu
from jax.experimental.pallas import tpu_sc as plsc

_DIM = 1024
_B = 4
_L = 2048
_NW = 32
_PPW = _L // _NW
_CH = 16
_NJ = _PPW // _CH
_NSTEP = _NJ * _B
_NB = 3
_LANES = 16


def _emb_body(ids_hbm, tok_hbm, pos_hbm, out_hbm, idx_v, pos_v, rows_v,
              gsem, osem, psem):
    nc = 2
    wid = lax.axis_index("s") * nc + lax.axis_index("c")
    p0 = wid * _PPW

    pos_cp = pltpu.async_copy(pos_hbm.at[pl.ds(p0, _PPW)], pos_v, psem)
    for bb in range(_B):
        pltpu.sync_copy(ids_hbm.at[pl.ds(bb * _L + p0, _PPW)],
                        idx_v.at[bb])

    def start_gather(s):
        j, bb = divmod(s, _B)
        rb = s % _NB
        return pltpu.async_copy(
            tok_hbm.at[idx_v.at[bb, pl.ds(j * _CH, _CH)]],
            rows_v.at[rb], gsem.at[rb])

    def start_out(s):
        j, bb = divmod(s, _B)
        rb = s % _NB
        return pltpu.async_copy(
            rows_v.at[rb],
            out_hbm.at[pl.ds(bb * _L + p0 + j * _CH, _CH)], osem.at[rb])

    gat_d = [None] * _NSTEP
    out_d = [None] * _NSTEP
    for s in range(min(_NB - 1, _NSTEP)):
        gat_d[s] = start_gather(s)
    pos_cp.wait()

    for s in range(_NSTEP):
        j, bb = divmod(s, _B)
        rb = s % _NB
        gat_d[s].wait()

        jbase = j * _CH

        @plsc.parallel_loop(0, _CH)
        def _(r):
            @plsc.parallel_loop(0, _DIM, step=_LANES, unroll=4)
            def _(co):
                sl = pl.ds(co, _LANES)
                plsc.addupdate(rows_v.at[rb, r, sl], pos_v[jbase + r, sl])

        out_d[s] = start_out(s)
        if s + _NB - 1 < _NSTEP:
            if s >= 1:
                out_d[s - 1].wait()
            gat_d[s + _NB - 1] = start_gather(s + _NB - 1)

    for s in range(_NSTEP - _NB, _NSTEP):
        out_d[s].wait()


@jax.jit
def _emb(ids_flat, tok_table, pos_table):
    mesh = plsc.VectorSubcoreMesh(core_axis_name="c", subcore_axis_name="s")
    return pl.kernel(
        _emb_body,
        out_type=jax.ShapeDtypeStruct((_B * _L, _DIM), jnp.float32),
        mesh=mesh,
        scratch_types=[
            pltpu.VMEM((_B, _PPW), jnp.int32),
            pltpu.VMEM((_PPW, _DIM), jnp.float32),
            pltpu.VMEM((_NB, _CH, _DIM), jnp.float32),
            pltpu.SemaphoreType.DMA((_NB,)),
            pltpu.SemaphoreType.DMA((_NB,)),
            pltpu.SemaphoreType.DMA,
        ],
    )(ids_flat, tok_table, pos_table)


def kernel(token_ids, tok_table, pos_table):
    b, l = token_ids.shape
    dim = tok_table.shape[1]
    ids_flat = token_ids.reshape(b * l).astype(jnp.int32)
    out = _emb(ids_flat, tok_table, pos_table)
    return out.reshape(b, l, dim)


# speedup vs baseline: 1.6654x; 1.0319x over previous
import jax
import jax.numpy as jnp
from jax import lax
from jax.experimental import pallas as pl
from jax.experimental.pallas import tpu as pltpu
from jax.experimental.pallas import tpu_sc as plsc

_DIM = 1024
_B = 4
_L = 2048
_NW = 32
_PPW = _L // _NW
_CH = 16
_NJ = _PPW // _CH
_NSTEP = _NJ * _B
_NB = 5
_LANES = 16


def _emb_body(ids_hbm, tok_hbm, pos_hbm, out_hbm, idx_v, pos_v, rows_v,
              gsem, osem, psem, isem):
    nc = 2
    wid = lax.axis_index("s") * nc + lax.axis_index("c")
    p0 = wid * _PPW

    idx_d = [
        pltpu.async_copy(ids_hbm.at[pl.ds(bb * _L + p0, _PPW)],
                         idx_v.at[bb], isem)
        for bb in range(_B)
    ]
    pos_d = [None] * _NJ
    for j in range(2):
        pos_d[j] = pltpu.async_copy(
            pos_hbm.at[pl.ds(p0 + j * _CH, _CH)], pos_v.at[j % 2],
            psem.at[j % 2])
    for d in idx_d:
        d.wait()

    def start_gather(s):
        j, bb = divmod(s, _B)
        rb = s % _NB
        return pltpu.async_copy(
            tok_hbm.at[idx_v.at[bb, pl.ds(j * _CH, _CH)]],
            rows_v.at[rb], gsem.at[rb])

    def start_out(s):
        j, bb = divmod(s, _B)
        rb = s % _NB
        return pltpu.async_copy(
            rows_v.at[rb],
            out_hbm.at[pl.ds(bb * _L + p0 + j * _CH, _CH)], osem.at[rb])

    gat_d = [None] * _NSTEP
    out_d = [None] * _NSTEP
    for s in range(_NB - 1):
        gat_d[s] = start_gather(s)

    for s in range(_NSTEP):
        j, bb = divmod(s, _B)
        rb = s % _NB
        if bb == 0:
            pos_d[j].wait()
        gat_d[s].wait()

        pb = j % 2

        @plsc.parallel_loop(0, _CH)
        def _(r):
            @plsc.parallel_loop(0, _DIM, step=_LANES, unroll=4)
            def _(co):
                sl = pl.ds(co, _LANES)
                plsc.addupdate(rows_v.at[rb, r, sl], pos_v[pb, r, sl])

        if bb == _B - 1 and j + 2 < _NJ:
            pos_d[j + 2] = pltpu.async_copy(
                pos_hbm.at[pl.ds(p0 + (j + 2) * _CH, _CH)],
                pos_v.at[j % 2], psem.at[j % 2])
        out_d[s] = start_out(s)
        if s + _NB - 1 < _NSTEP:
            if s >= 1:
                out_d[s - 1].wait()
            gat_d[s + _NB - 1] = start_gather(s + _NB - 1)

    for s in range(_NSTEP - _NB, _NSTEP):
        out_d[s].wait()


@jax.jit
def _emb(ids_flat, tok_table, pos_table):
    mesh = plsc.VectorSubcoreMesh(core_axis_name="c", subcore_axis_name="s")
    return pl.kernel(
        _emb_body,
        out_type=jax.ShapeDtypeStruct((_B * _L, _DIM), jnp.float32),
        mesh=mesh,
        scratch_types=[
            pltpu.VMEM((_B, _PPW), jnp.int32),
            pltpu.VMEM((2, _CH, _DIM), jnp.float32),
            pltpu.VMEM((_NB, _CH, _DIM), jnp.float32),
            pltpu.SemaphoreType.DMA((_NB,)),
            pltpu.SemaphoreType.DMA((_NB,)),
            pltpu.SemaphoreType.DMA((2,)),
            pltpu.SemaphoreType.DMA,
        ],
    )(ids_flat, tok_table, pos_table)


def kernel(token_ids, tok_table, pos_table):
    b, l = token_ids.shape
    dim = tok_table.shape[1]
    ids_flat = token_ids.reshape(b * l).astype(jnp.int32)
    out = _emb(ids_flat, tok_table, pos_table)
    return out.reshape(b, l, dim)
